# baseline (speedup 1.0000x reference)
import jax
import jax.numpy as jnp
from jax import lax
from jax.experimental import pallas as pl
from jax.experimental.pallas import tpu as pltpu
from jax.experimental.pallas import tpu_sc as plsc

N = 10000
E = 320000
D = 128

NPAD = 10240
NROWS = NPAD // 128
NC = 2
NS = 16
NW = NC * NS
EROWS = E // 128
RPT = EROWS // NW
REM = EROWS - NW * RPT
RMAX = RPT + 1
GRP = 13
SLICE = NPAD // NS


def _rowsum_body(x_ref, o_ref):
    o_ref[...] = jnp.sum(x_ref[...], axis=1).reshape(16, 128)


_rowsum = pl.pallas_call(
    _rowsum_body,
    grid=(NPAD // 2048,),
    in_specs=[pl.BlockSpec((2048, D), lambda i: (i, 0))],
    out_specs=pl.BlockSpec((16, 128), lambda i: (i, 0)),
    out_shape=jax.ShapeDtypeStruct((NROWS, 128), jnp.float32),
)


def _edge_body(s_hbm, sd_hbm, w_hbm, t0_out, t1_out, c0_out, c1_out,
               s_v, sd_v, w_v, msg_v, ones_v, zb_v,
               t_sh, c_sh, sem, scat_sem):
    cid = lax.axis_index("c")
    sid = lax.axis_index("s")
    wid = sid * NC + cid

    r_lo = RPT * wid + jnp.minimum(wid, REM)
    start = jnp.minimum(r_lo, EROWS - RMAX)
    roff = r_lo - start

    cps = [
        pltpu.async_copy(s_hbm, s_v, sem),
        pltpu.async_copy(sd_hbm.at[pl.ds(start, RMAX)], sd_v, sem),
        pltpu.async_copy(w_hbm.at[pl.ds(start * 128, RMAX * 128)], w_v, sem),
    ]

    for i in range(128 // 16):
        ones_v[pl.ds(i * 16, 16)] = jnp.ones((16,), jnp.float32)

    def zbody(i, c):
        zb_v[pl.ds(i * 16, 16)] = jnp.zeros((16,), jnp.float32)
        return c
    lax.fori_loop(0, SLICE // 16, zbody, 0)
    pltpu.sync_copy(zb_v, t_sh.at[pl.ds(sid * SLICE, SLICE)])
    pltpu.sync_copy(zb_v, c_sh.at[pl.ds(sid * SLICE, SLICE)])
    for cp in cps:
        cp.wait()
    plsc.subcore_barrier()

    def scat_pair(r):
        c1 = pltpu.make_async_copy(msg_v.at[pl.ds(r * 128, 128)],
                                   t_sh.at[sd_v.at[r, 1]], scat_sem)
        c2 = pltpu.make_async_copy(ones_v, c_sh.at[sd_v.at[r, 1]], scat_sem)
        return c1, c2

    def do_row(r):
        srcs = [sd_v[r, 0, pl.ds(k * 16, 16)] for k in range(8)]
        vals = [plsc.load_gather(s_v, [lax.shift_right_logical(s16, 7),
                                       lax.bitwise_and(s16, 127)])
                for s16 in srcs]
        ws = [w_v[pl.ds(r * 128 + k * 16, 16)] for k in range(8)]
        for k in range(8):
            msg_v[pl.ds(r * 128 + k * 16, 16)] = vals[k] * ws[k]
        c1, c2 = scat_pair(r)
        c1.start(add=True)
        c2.start(add=True)

    def gbody(g, c):
        for i in range(GRP):
            do_row(roff + g * GRP + i)

        @pl.when(g > 0)
        def _():
            for i in range(GRP):
                p1, p2 = scat_pair(roff + (g - 1) * GRP + i)
                p1.wait()
                p2.wait()
        return c
    lax.fori_loop(0, RPT // GRP, gbody, 0)
    for i in range(GRP):
        f1, f2 = scat_pair(roff + RPT - GRP + i)
        f1.wait()
        f2.wait()

    @pl.when(wid < REM)
    def _():
        do_row(roff + RPT)
        e1, e2 = scat_pair(roff + RPT)
        e1.wait()
        e2.wait()
    plsc.subcore_barrier()

    off = sid * SLICE

    @pl.when(cid == 0)
    def _():
        cpo1 = pltpu.async_copy(t_sh.at[pl.ds(off, SLICE)],
                                t0_out.at[pl.ds(off, SLICE)], sem)
        cpo2 = pltpu.async_copy(c_sh.at[pl.ds(off, SLICE)],
                                c0_out.at[pl.ds(off, SLICE)], sem)
        cpo1.wait()
        cpo2.wait()

    @pl.when(cid == 1)
    def _():
        cpo1 = pltpu.async_copy(t_sh.at[pl.ds(off, SLICE)],
                                t1_out.at[pl.ds(off, SLICE)], sem)
        cpo2 = pltpu.async_copy(c_sh.at[pl.ds(off, SLICE)],
                                c1_out.at[pl.ds(off, SLICE)], sem)
        cpo1.wait()
        cpo2.wait()


_edge_call = pl.kernel(
    _edge_body,
    out_type=[jax.ShapeDtypeStruct((NPAD,), jnp.float32)] * 4,
    mesh=plsc.VectorSubcoreMesh(core_axis_name="c", subcore_axis_name="s",
                                num_cores=NC, num_subcores=NS),
    scratch_types=[
        pltpu.VMEM((NROWS, 128), jnp.float32),
        pltpu.VMEM((RMAX, 2, 128), jnp.int32),
        pltpu.VMEM((RMAX * 128,), jnp.float32),
        pltpu.VMEM((RMAX * 128,), jnp.float32),
        pltpu.VMEM((128,), jnp.float32),
        pltpu.VMEM((SLICE,), jnp.float32),
        pltpu.VMEM_SHARED((NPAD,), jnp.float32),
        pltpu.VMEM_SHARED((NPAD,), jnp.float32),
        pltpu.SemaphoreType.DMA,
        pltpu.SemaphoreType.DMA,
    ],
    compiler_params=pltpu.CompilerParams(needs_layout_passes=False),
)


def _final_body(s_ref, t0_ref, t1_ref, c0_ref, c1_ref, o_ref):
    t = t0_ref[...] + t1_ref[...]
    c = jnp.maximum(c0_ref[...] + c1_ref[...], 1.0)
    o_ref[...] = jnp.abs(s_ref[...] - t / c)


_final = pl.pallas_call(
    _final_body,
    out_shape=jax.ShapeDtypeStruct((NROWS, 128), jnp.float32),
)


def kernel(x, edge_index, edge_weights):
    s2d = _rowsum(x)
    sd = edge_index.reshape(2, EROWS, 128).transpose(1, 0, 2)

    t0, t1, c0, c1 = _edge_call(s2d, sd, edge_weights)

    info2d = _final(s2d, t0.reshape(NROWS, 128), t1.reshape(NROWS, 128),
                    c0.reshape(NROWS, 128), c1.reshape(NROWS, 128))
    return info2d.reshape(NPAD)[:N]


# speedup vs baseline: 1.0095x; 1.0095x over previous
import jax
import jax.numpy as jnp
from jax import lax
from jax.experimental import pallas as pl
from jax.experimental.pallas import tpu as pltpu
from jax.experimental.pallas import tpu_sc as plsc

N = 10000
E = 320000
D = 128

NPAD = 10240
NROWS = NPAD // 128
NC = 2
NS = 16
NW = NC * NS
EROWS = E // 128
RPT = EROWS // NW
REM = EROWS - NW * RPT
RMAX = RPT + 1
GRP = 6
SLICE = NPAD // NS


def _rowsum_body(x_ref, o_ref):
    o_ref[...] = jnp.sum(x_ref[...], axis=1).reshape(16, 128)


_rowsum = pl.pallas_call(
    _rowsum_body,
    grid=(NPAD // 2048,),
    in_specs=[pl.BlockSpec((2048, D), lambda i: (i, 0))],
    out_specs=pl.BlockSpec((16, 128), lambda i: (i, 0)),
    out_shape=jax.ShapeDtypeStruct((NROWS, 128), jnp.float32),
)


def _edge_body(s_hbm, sd_hbm, w_hbm, t0_out, t1_out, c0_out, c1_out,
               s_v, sd_v, w_v, msg_v, ones_v, zb_v,
               t_sh, c_sh, sem, scat_sem):
    cid = lax.axis_index("c")
    sid = lax.axis_index("s")
    wid = sid * NC + cid

    r_lo = RPT * wid + jnp.minimum(wid, REM)
    start = jnp.minimum(r_lo, EROWS - RMAX)
    roff = r_lo - start

    cps = [
        pltpu.async_copy(s_hbm, s_v, sem),
        pltpu.async_copy(sd_hbm.at[pl.ds(start, RMAX)], sd_v, sem),
        pltpu.async_copy(w_hbm.at[pl.ds(start * 128, RMAX * 128)], w_v, sem),
    ]

    for i in range(128 // 16):
        ones_v[pl.ds(i * 16, 16)] = jnp.ones((16,), jnp.float32)

    def zbody(i, c):
        zb_v[pl.ds(i * 16, 16)] = jnp.zeros((16,), jnp.float32)
        return c
    lax.fori_loop(0, SLICE // 16, zbody, 0)
    pltpu.sync_copy(zb_v, t_sh.at[pl.ds(sid * SLICE, SLICE)])
    pltpu.sync_copy(zb_v, c_sh.at[pl.ds(sid * SLICE, SLICE)])
    for cp in cps:
        cp.wait()
    plsc.subcore_barrier()

    def scat_pair(r):
        c1 = pltpu.make_async_copy(msg_v.at[pl.ds(r * 128, 128)],
                                   t_sh.at[sd_v.at[r, 1]], scat_sem)
        c2 = pltpu.make_async_copy(ones_v, c_sh.at[sd_v.at[r, 1]], scat_sem)
        return c1, c2

    def do_row(r):
        srcs = [sd_v[r, 0, pl.ds(k * 16, 16)] for k in range(8)]
        vals = [plsc.load_gather(s_v, [lax.shift_right_logical(s16, 7),
                                       lax.bitwise_and(s16, 127)])
                for s16 in srcs]
        ws = [w_v[pl.ds(r * 128 + k * 16, 16)] for k in range(8)]
        for k in range(8):
            msg_v[pl.ds(r * 128 + k * 16, 16)] = vals[k] * ws[k]
        c1, c2 = scat_pair(r)
        c1.start(add=True)
        c2.start(add=True)

    def gbody(g, c):
        for i in range(GRP):
            do_row(roff + g * GRP + i)

        @pl.when(g > 0)
        def _():
            for i in range(GRP):
                p1, p2 = scat_pair(roff + (g - 1) * GRP + i)
                p1.wait()
                p2.wait()
        return c
    lax.fori_loop(0, RPT // GRP, gbody, 0)
    for i in range(GRP):
        f1, f2 = scat_pair(roff + RPT - GRP + i)
        f1.wait()
        f2.wait()

    @pl.when(wid < REM)
    def _():
        do_row(roff + RPT)
        e1, e2 = scat_pair(roff + RPT)
        e1.wait()
        e2.wait()
    plsc.subcore_barrier()

    off = sid * SLICE

    @pl.when(cid == 0)
    def _():
        cpo1 = pltpu.async_copy(t_sh.at[pl.ds(off, SLICE)],
                                t0_out.at[pl.ds(off, SLICE)], sem)
        cpo2 = pltpu.async_copy(c_sh.at[pl.ds(off, SLICE)],
                                c0_out.at[pl.ds(off, SLICE)], sem)
        cpo1.wait()
        cpo2.wait()

    @pl.when(cid == 1)
    def _():
        cpo1 = pltpu.async_copy(t_sh.at[pl.ds(off, SLICE)],
                                t1_out.at[pl.ds(off, SLICE)], sem)
        cpo2 = pltpu.async_copy(c_sh.at[pl.ds(off, SLICE)],
                                c1_out.at[pl.ds(off, SLICE)], sem)
        cpo1.wait()
        cpo2.wait()


_edge_call = pl.kernel(
    _edge_body,
    out_type=[jax.ShapeDtypeStruct((NPAD,), jnp.float32)] * 4,
    mesh=plsc.VectorSubcoreMesh(core_axis_name="c", subcore_axis_name="s",
                                num_cores=NC, num_subcores=NS),
    scratch_types=[
        pltpu.VMEM((NROWS, 128), jnp.float32),
        pltpu.VMEM((RMAX, 2, 128), jnp.int32),
        pltpu.VMEM((RMAX * 128,), jnp.float32),
        pltpu.VMEM((RMAX * 128,), jnp.float32),
        pltpu.VMEM((128,), jnp.float32),
        pltpu.VMEM((SLICE,), jnp.float32),
        pltpu.VMEM_SHARED((NPAD,), jnp.float32),
        pltpu.VMEM_SHARED((NPAD,), jnp.float32),
        pltpu.SemaphoreType.DMA,
        pltpu.SemaphoreType.DMA,
    ],
    compiler_params=pltpu.CompilerParams(needs_layout_passes=False),
)


def _final_body(s_ref, t0_ref, t1_ref, c0_ref, c1_ref, o_ref):
    t = t0_ref[...] + t1_ref[...]
    c = jnp.maximum(c0_ref[...] + c1_ref[...], 1.0)
    o_ref[...] = jnp.abs(s_ref[...] - t / c)


_final = pl.pallas_call(
    _final_body,
    out_shape=jax.ShapeDtypeStruct((NROWS, 128), jnp.float32),
)


def kernel(x, edge_index, edge_weights):
    s2d = _rowsum(x)
    sd = edge_index.reshape(2, EROWS, 128).transpose(1, 0, 2)

    t0, t1, c0, c1 = _edge_call(s2d, sd, edge_weights)

    info2d = _final(s2d, t0.reshape(NROWS, 128), t1.reshape(NROWS, 128),
                    c0.reshape(NROWS, 128), c1.reshape(NROWS, 128))
    return info2d.reshape(NPAD)[:N]
